# Initial kernel scaffold; baseline (speedup 1.0000x reference)
import jax
import jax.numpy as jnp
from jax.experimental import pallas as pl


def kernel(word_ids, chars_embeddings, word_table, pos_table, gamma, beta):
    raise NotImplementedError("write your pallas kernel here")


# speedup vs baseline: 1.0272x; 1.0272x over previous
import functools

import jax
import jax.numpy as jnp
from jax import lax
from jax.experimental import pallas as pl
from jax.experimental.pallas import tpu as pltpu
from jax.experimental.pallas import tpu_sc as plsc

NW = 32
KC = 128
NBUF = 2
EPS = 1e-12


def _gather_body(idx_hbm, table_hbm, out_hbm, idx_v, rows0, rows1,
                 sem0, sem1):
    nchunk, kc = idx_v.shape
    wid = lax.axis_index("s") * 2 + lax.axis_index("c")
    base = wid * nchunk * kc
    rows = (rows0, rows1)
    sems = (sem0, sem1)
    pltpu.sync_copy(idx_hbm.at[wid], idx_v)

    def start(g, b):
        pltpu.async_copy(table_hbm.at[idx_v.at[g]], rows[b], sems[b])

    def wait(b):
        pltpu.make_async_copy(table_hbm.at[idx_v.at[0]], rows[b],
                              sems[b]).wait()

    start(0, 0)

    def chunk_pair(g2, _):
        for b in range(2):
            g = g2 * 2 + b

            @pl.when(g + 1 < nchunk)
            def _():
                start(g + 1, 1 - b)

            wait(b)
            pltpu.sync_copy(rows[b], out_hbm.at[pl.ds(base + g * kc, kc)])
        return ()

    lax.fori_loop(0, nchunk // 2, chunk_pair, (), unroll=False)


def _sc_gather(idx, table):
    nw, nchunk, kc = idx.shape
    t = nw * nchunk * kc
    word_dim = table.shape[1]
    mesh = plsc.VectorSubcoreMesh(core_axis_name="c", subcore_axis_name="s",
                                  num_cores=2, num_subcores=16)
    return pl.kernel(
        _gather_body,
        out_type=jax.ShapeDtypeStruct((t, word_dim), jnp.float32),
        mesh=mesh,
        scratch_types=[
            pltpu.VMEM((nchunk, kc), jnp.int32),
            pltpu.VMEM((kc, word_dim), jnp.float32),
            pltpu.VMEM((kc, word_dim), jnp.float32),
            pltpu.SemaphoreType.DMA,
            pltpu.SemaphoreType.DMA,
        ],
    )(idx, table)


def _ln_body(words_ref, chars_ref, pos_ref, gamma_ref, beta_ref, out_ref):
    wd = out_ref.shape[-1] - chars_ref.shape[-1]
    x = jnp.concatenate([words_ref[..., :wd], chars_ref[...]], axis=-1)
    x = x + pos_ref[...][None, :, :]
    mu = jnp.mean(x, axis=-1, keepdims=True)
    var = jnp.mean(jnp.square(x - mu), axis=-1, keepdims=True)
    y = (x - mu) * lax.rsqrt(var + EPS)
    out_ref[...] = y * gamma_ref[...][None] + beta_ref[...][None]


def _tc_ln(words, chars, pos, gamma, beta):
    b, l, wdp = words.shape
    cd = chars.shape[-1]
    h = pos.shape[-1]
    rb = 8
    grid = (b // rb,)
    return pl.pallas_call(
        _ln_body,
        out_shape=jax.ShapeDtypeStruct((b, l, h), jnp.float32),
        grid=grid,
        in_specs=[
            pl.BlockSpec((rb, l, wdp), lambda i: (i, 0, 0)),
            pl.BlockSpec((rb, l, cd), lambda i: (i, 0, 0)),
            pl.BlockSpec((l, h), lambda i: (0, 0)),
            pl.BlockSpec((1, h), lambda i: (0, 0)),
            pl.BlockSpec((1, h), lambda i: (0, 0)),
        ],
        out_specs=pl.BlockSpec((rb, l, h), lambda i: (i, 0, 0)),
    )(words, chars, pos, gamma, beta)


@jax.jit
def kernel(word_ids, chars_embeddings, word_table, pos_table, gamma, beta):
    b, l = word_ids.shape
    idx = word_ids.reshape(NW, -1, KC).astype(jnp.int32)
    table_p = jnp.pad(word_table, ((0, 0), (0, 384 - word_table.shape[1])))
    words = _sc_gather(idx, table_p)
    out = _tc_ln(
        words.reshape(b, l, 384),
        chars_embeddings,
        pos_table[:l],
        gamma.reshape(1, -1),
        beta.reshape(1, -1),
    )
    return out


# speedup vs baseline: 1.3872x; 1.3505x over previous
import functools

import jax
import jax.numpy as jnp
from jax import lax
from jax.experimental import pallas as pl
from jax.experimental.pallas import tpu as pltpu
from jax.experimental.pallas import tpu_sc as plsc

NW = 32
KC = 128
NBUF = 2
EPS = 1e-12


def _gather_body(idx_hbm, table_hbm, out_hbm, idx_v, rows0, rows1,
                 sem0, sem1):
    nchunk, kc = idx_v.shape
    wid = lax.axis_index("s") * 2 + lax.axis_index("c")
    base = wid * nchunk * kc
    rows = (rows0, rows1)
    sems = (sem0, sem1)
    pltpu.sync_copy(idx_hbm.at[wid], idx_v)

    def start(g, b):
        pltpu.async_copy(table_hbm.at[idx_v.at[g]], rows[b], sems[b])

    def wait(b):
        pltpu.make_async_copy(table_hbm.at[idx_v.at[0]], rows[b],
                              sems[b]).wait()

    start(0, 0)

    def chunk_pair(g2, _):
        for b in range(2):
            g = g2 * 2 + b

            @pl.when(g + 1 < nchunk)
            def _():
                start(g + 1, 1 - b)

            wait(b)
            pltpu.sync_copy(rows[b], out_hbm.at[pl.ds(base + g * kc, kc)])
        return ()

    lax.fori_loop(0, nchunk // 2, chunk_pair, (), unroll=False)


def _sc_gather(idx, table):
    nw, nchunk, kc = idx.shape
    t = nw * nchunk * kc
    word_dim = table.shape[1]
    mesh = plsc.VectorSubcoreMesh(core_axis_name="c", subcore_axis_name="s",
                                  num_cores=2, num_subcores=16)
    return pl.kernel(
        _gather_body,
        out_type=jax.ShapeDtypeStruct((t, word_dim), jnp.float32),
        mesh=mesh,
        scratch_types=[
            pltpu.VMEM((nchunk, kc), jnp.int32),
            pltpu.VMEM((kc, word_dim), jnp.float32),
            pltpu.VMEM((kc, word_dim), jnp.float32),
            pltpu.SemaphoreType.DMA,
            pltpu.SemaphoreType.DMA,
        ],
    )(idx, table)


def _pad_body(src_ref, out_ref):
    rb, wd = src_ref.shape
    wdp = out_ref.shape[-1]
    out_ref[...] = jnp.pad(src_ref[...], ((0, 0), (0, wdp - wd)))


def _tc_pad(table, wdp):
    v, wd = table.shape
    rb = 2000
    return pl.pallas_call(
        _pad_body,
        out_shape=jax.ShapeDtypeStruct((v, wdp), jnp.float32),
        grid=(v // rb,),
        in_specs=[pl.BlockSpec((rb, wd), lambda i: (i, 0))],
        out_specs=pl.BlockSpec((rb, wdp), lambda i: (i, 0)),
    )(table)


def _ln_body(words_ref, chars_ref, pos_ref, gamma_ref, beta_ref, out_ref):
    wd = out_ref.shape[-1] - chars_ref.shape[-1]
    x = jnp.concatenate([words_ref[..., :wd], chars_ref[...]], axis=-1)
    x = x + pos_ref[...][None, :, :]
    mu = jnp.mean(x, axis=-1, keepdims=True)
    var = jnp.mean(jnp.square(x - mu), axis=-1, keepdims=True)
    y = (x - mu) * lax.rsqrt(var + EPS)
    out_ref[...] = y * gamma_ref[...][None] + beta_ref[...][None]


def _tc_ln(words, chars, pos, gamma, beta):
    b, l, wdp = words.shape
    cd = chars.shape[-1]
    h = pos.shape[-1]
    rb = 8
    grid = (b // rb,)
    return pl.pallas_call(
        _ln_body,
        out_shape=jax.ShapeDtypeStruct((b, l, h), jnp.float32),
        grid=grid,
        in_specs=[
            pl.BlockSpec((rb, l, wdp), lambda i: (i, 0, 0)),
            pl.BlockSpec((rb, l, cd), lambda i: (i, 0, 0)),
            pl.BlockSpec((l, h), lambda i: (0, 0)),
            pl.BlockSpec((1, h), lambda i: (0, 0)),
            pl.BlockSpec((1, h), lambda i: (0, 0)),
        ],
        out_specs=pl.BlockSpec((rb, l, h), lambda i: (i, 0, 0)),
    )(words, chars, pos, gamma, beta)


@jax.jit
def kernel(word_ids, chars_embeddings, word_table, pos_table, gamma, beta):
    b, l = word_ids.shape
    idx = word_ids.reshape(NW, -1, KC).astype(jnp.int32)
    table_p = _tc_pad(word_table, 384)
    words = _sc_gather(idx, table_p)
    out = _tc_ln(
        words.reshape(b, l, 384),
        chars_embeddings,
        pos_table[:l],
        gamma.reshape(1, -1),
        beta.reshape(1, -1),
    )
    return out


# speedup vs baseline: 1.4248x; 1.0271x over previous
import functools

import jax
import jax.numpy as jnp
from jax import lax
from jax.experimental import pallas as pl
from jax.experimental.pallas import tpu as pltpu
from jax.experimental.pallas import tpu_sc as plsc

NW = 32
KC = 128
NBUF = 2
EPS = 1e-12


def _gather_body(idx_hbm, table_hbm, out_hbm, idx_v, rows0, rows1,
                 sem0, sem1):
    nchunk, kc = idx_v.shape
    wid = lax.axis_index("s") * 2 + lax.axis_index("c")
    base = wid * nchunk * kc
    rows = (rows0, rows1)
    sems = (sem0, sem1)
    pltpu.sync_copy(idx_hbm.at[wid], idx_v)

    def start(g, b):
        pltpu.async_copy(table_hbm.at[idx_v.at[g]], rows[b], sems[b])

    def wait(b):
        pltpu.make_async_copy(table_hbm.at[idx_v.at[0]], rows[b],
                              sems[b]).wait()

    start(0, 0)

    def chunk_pair(g2, _):
        for b in range(2):
            g = g2 * 2 + b

            @pl.when(g + 1 < nchunk)
            def _():
                start(g + 1, 1 - b)

            wait(b)
            pltpu.sync_copy(rows[b], out_hbm.at[pl.ds(base + g * kc, kc)])
        return ()

    lax.fori_loop(0, nchunk // 2, chunk_pair, (), unroll=False)


def _sc_gather(idx, table):
    nw, nchunk, kc = idx.shape
    t = nw * nchunk * kc
    word_dim = table.shape[1]
    mesh = plsc.VectorSubcoreMesh(core_axis_name="c", subcore_axis_name="s",
                                  num_cores=2, num_subcores=16)
    return pl.kernel(
        _gather_body,
        out_type=jax.ShapeDtypeStruct((t, word_dim), jnp.float32),
        mesh=mesh,
        scratch_types=[
            pltpu.VMEM((nchunk, kc), jnp.int32),
            pltpu.VMEM((kc, word_dim), jnp.float32),
            pltpu.VMEM((kc, word_dim), jnp.float32),
            pltpu.SemaphoreType.DMA,
            pltpu.SemaphoreType.DMA,
        ],
    )(idx, table)


def _pad_body(src_ref, out_ref):
    rb, wd = src_ref.shape
    wdp = out_ref.shape[-1]
    out_ref[...] = jnp.pad(src_ref[...], ((0, 0), (0, wdp - wd)))


def _tc_pad(table, wdp):
    v, wd = table.shape
    rb = 2000
    return pl.pallas_call(
        _pad_body,
        out_shape=jax.ShapeDtypeStruct((v, wdp), jnp.float32),
        grid=(v // rb,),
        in_specs=[pl.BlockSpec((rb, wd), lambda i: (i, 0))],
        out_specs=pl.BlockSpec((rb, wdp), lambda i: (i, 0)),
    )(table)


def _ln_body(words_ref, chars_ref, pos_ref, gamma_ref, beta_ref, out_ref):
    wd = out_ref.shape[-1] - chars_ref.shape[-1]
    x = jnp.concatenate([words_ref[..., :wd], chars_ref[...]], axis=-1)
    x = x + pos_ref[...][None, :, :]
    mu = jnp.mean(x, axis=-1, keepdims=True)
    var = jnp.mean(jnp.square(x - mu), axis=-1, keepdims=True)
    y = (x - mu) * lax.rsqrt(var + EPS)
    out_ref[...] = y * gamma_ref[...][None] + beta_ref[...][None]


def _tc_ln(words, chars, pos, gamma, beta):
    b, l, wdp = words.shape
    cd = chars.shape[-1]
    h = pos.shape[-1]
    rb = 16
    grid = (b // rb,)
    return pl.pallas_call(
        _ln_body,
        out_shape=jax.ShapeDtypeStruct((b, l, h), jnp.float32),
        grid=grid,
        in_specs=[
            pl.BlockSpec((rb, l, wdp), lambda i: (i, 0, 0)),
            pl.BlockSpec((rb, l, cd), lambda i: (i, 0, 0)),
            pl.BlockSpec((l, h), lambda i: (0, 0)),
            pl.BlockSpec((1, h), lambda i: (0, 0)),
            pl.BlockSpec((1, h), lambda i: (0, 0)),
        ],
        out_specs=pl.BlockSpec((rb, l, h), lambda i: (i, 0, 0)),
    )(words, chars, pos, gamma, beta)


@jax.jit
def kernel(word_ids, chars_embeddings, word_table, pos_table, gamma, beta):
    b, l = word_ids.shape
    idx = word_ids.reshape(NW, -1, KC).astype(jnp.int32)
    table_p = _tc_pad(word_table, 384)
    words = _sc_gather(idx, table_p)
    out = _tc_ln(
        words.reshape(b, l, 384),
        chars_embeddings,
        pos_table[:l],
        gamma.reshape(1, -1),
        beta.reshape(1, -1),
    )
    return out
